# Initial kernel scaffold; baseline (speedup 1.0000x reference)
import jax
import jax.numpy as jnp
from jax.experimental import pallas as pl


def kernel(x, emb_table):
    raise NotImplementedError("write your pallas kernel here")


# speedup vs baseline: 1.3675x; 1.3675x over previous
import functools

import numpy as np
import jax
import jax.numpy as jnp
from jax import lax
from jax.experimental import pallas as pl
from jax.experimental.pallas import tpu as pltpu
from jax.experimental.pallas import tpu_sc as plsc

_VOCAB = 100000
_D = 1024
_SEQ = 4096
_BATCH = 4
_SCALE = 32.0

_NC, _NS = 2, 16
_NW = _NC * _NS
_S_PER_W = _SEQ // _NW
_R = 32
_CHUNKS = _S_PER_W // _R
_LANES = 16


def _pe_table() -> np.ndarray:
    pos = np.arange(_SEQ, dtype=np.float32)[:, None]
    div = np.exp(np.arange(0, _D, 2, dtype=np.float32) * (-np.log(10000.0) / _D))
    pe = np.zeros((_SEQ, _D), dtype=np.float32)
    pe[:, 0::2] = np.sin(pos * div)
    pe[:, 1::2] = np.cos(pos * div)
    return pe


_PE = _pe_table()

_mesh = plsc.VectorSubcoreMesh(core_axis_name="c", subcore_axis_name="s")


@functools.partial(
    pl.kernel,
    out_type=jax.ShapeDtypeStruct((_BATCH * _SEQ, _D), jnp.float32),
    mesh=_mesh,
    scratch_types=[
        pltpu.VMEM((_R,), jnp.int32),
        pltpu.VMEM((_R, _D), jnp.float32),
        pltpu.VMEM((_R, _D), jnp.float32),
        pltpu.SemaphoreType.DMA,
    ],
)
def _emb_kernel(x_hbm, table_hbm, pe_hbm, out_hbm, idx_v, rows_v, pe_v, sem):
    wid = lax.axis_index("s") * _NC + lax.axis_index("c")
    s_base = wid * _S_PER_W

    def chunk_body(c, carry):
        s0 = s_base + c * _R
        pltpu.sync_copy(pe_hbm.at[pl.ds(s0, _R)], pe_v)

        def batch_body(b, carry2):
            row0 = b * _SEQ + s0
            pltpu.sync_copy(x_hbm.at[pl.ds(row0, _R)], idx_v)
            pltpu.async_copy(table_hbm.at[idx_v], rows_v, sem).wait()

            def row_body(r, carry3):
                for j in range(_D // _LANES):
                    sl = pl.ds(j * _LANES, _LANES)
                    rows_v[r, sl] = rows_v[r, sl] * _SCALE + pe_v[r, sl]
                return carry3

            lax.fori_loop(0, _R, row_body, 0, unroll=False)
            pltpu.sync_copy(rows_v, out_hbm.at[pl.ds(row0, _R)])
            return carry2

        lax.fori_loop(0, _BATCH, batch_body, 0, unroll=False)
        return carry

    lax.fori_loop(0, _CHUNKS, chunk_body, 0, unroll=False)


def kernel(x, emb_table):
    xf = x.reshape(-1).astype(jnp.int32)
    out = _emb_kernel(xf, emb_table, jnp.asarray(_PE))
    return out.reshape(_BATCH, _SEQ, _D)


# speedup vs baseline: 1.5315x; 1.1200x over previous
import functools

import numpy as np
import jax
import jax.numpy as jnp
from jax import lax
from jax.experimental import pallas as pl
from jax.experimental.pallas import tpu as pltpu
from jax.experimental.pallas import tpu_sc as plsc

_VOCAB = 100000
_D = 1024
_SEQ = 4096
_BATCH = 4
_SCALE = 32.0

_NC, _NS = 2, 16
_NW = _NC * _NS
_S_PER_W = _SEQ // _NW
_R = 16
_CPW = _S_PER_W // _R
_ROUNDS = _CPW * _BATCH
_LANES = 16
_NSLOT = 4


def _pe_table() -> np.ndarray:
    pos = np.arange(_SEQ, dtype=np.float32)[:, None]
    div = np.exp(np.arange(0, _D, 2, dtype=np.float32) * (-np.log(10000.0) / _D))
    pe = np.zeros((_SEQ, _D), dtype=np.float32)
    pe[:, 0::2] = np.sin(pos * div)
    pe[:, 1::2] = np.cos(pos * div)
    return pe


_PE = _pe_table()

_mesh = plsc.VectorSubcoreMesh(core_axis_name="c", subcore_axis_name="s")


@functools.partial(
    pl.kernel,
    out_type=jax.ShapeDtypeStruct((_BATCH * _SEQ, _D), jnp.float32),
    mesh=_mesh,
    scratch_types=[
        pltpu.VMEM((_BATCH * _S_PER_W,), jnp.int32),
        pltpu.VMEM((_NSLOT, _R, _D), jnp.float32),
        pltpu.VMEM((2, _R, _D), jnp.float32),
        pltpu.SemaphoreType.DMA,
        pltpu.SemaphoreType.DMA,
        pltpu.SemaphoreType.DMA,
        pltpu.SemaphoreType.DMA,
        pltpu.SemaphoreType.DMA,
        pltpu.SemaphoreType.DMA,
        pltpu.SemaphoreType.DMA,
        pltpu.SemaphoreType.DMA,
        pltpu.SemaphoreType.DMA,
        pltpu.SemaphoreType.DMA,
    ],
)
def _emb_kernel(x_hbm, table_hbm, pe_hbm, out_hbm, idx_v, rows_v, pe_v,
                g0, g1, g2, g3, o0, o1, o2, o3, p0, p1):
    gsem = [g0, g1, g2, g3]
    osem = [o0, o1, o2, o3]
    psem = [p0, p1]

    wid = lax.axis_index("s") * _NC + lax.axis_index("c")
    s_base = wid * _S_PER_W

    for b in range(_BATCH):
        pltpu.sync_copy(x_hbm.at[pl.ds(b * _SEQ + s_base, _S_PER_W)],
                        idx_v.at[pl.ds(b * _S_PER_W, _S_PER_W)])

    def idx_slice(c, b):
        return idx_v.at[pl.ds(b * _S_PER_W + c * _R, _R)]

    def start_gather(c, b, slot):
        return pltpu.async_copy(table_hbm.at[idx_slice(c, b)],
                                rows_v.at[slot], gsem[slot])

    def wait_gather(c, b, slot):
        pltpu.make_async_copy(table_hbm.at[idx_slice(c, b)],
                              rows_v.at[slot], gsem[slot]).wait()

    def out_rows(c, b):
        return out_hbm.at[pl.ds(b * _SEQ + s_base + c * _R, _R)]

    def start_out(c, b, slot):
        return pltpu.async_copy(rows_v.at[slot], out_rows(c, b), osem[slot])

    def wait_out(c, b, slot):
        pltpu.make_async_copy(rows_v.at[slot], out_rows(c, b),
                              osem[slot]).wait()

    def pe_rows(c):
        return pe_hbm.at[pl.ds(s_base + c * _R, _R)]

    def start_pe(c, par):
        return pltpu.async_copy(pe_rows(c), pe_v.at[par], psem[par])

    def wait_pe(c, par):
        pltpu.make_async_copy(pe_rows(c), pe_v.at[par], psem[par]).wait()

    start_pe(0, 0)
    start_gather(0, 0, 0)
    start_gather(0, 1, 1)

    def group_body(gg, carry):
        for k in range(8):
            r = gg * 8 + k
            c = 2 * gg + k // 4
            b = k % 4
            slot = k % 4
            par = (k // 4) % 2

            if k % 4 == 0:
                wait_pe(c, par)

                @pl.when(c < _CPW - 1)
                def _():
                    start_pe(c + 1, 1 - par)

            wait_gather(c, b, slot)

            def row_body(rr, car):
                for j in range(_D // _LANES):
                    sl = pl.ds(j * _LANES, _LANES)
                    rows_v[slot, rr, sl] = (rows_v[slot, rr, sl] * _SCALE
                                            + pe_v[par, rr, sl])
                return car

            lax.fori_loop(0, _R, row_body, 0, unroll=False)

            start_out(c, b, slot)

            k2 = k + 2
            c2 = 2 * gg + k2 // 4
            b2 = k2 % 4
            slot2 = k2 % 4
            r2 = r + 2

            @pl.when(r2 < _ROUNDS)
            def _():
                @pl.when(r >= 2)
                def _():
                    wait_out(c2, b2, slot2)

                start_gather(c2, b2, slot2)

        return carry

    lax.fori_loop(0, _ROUNDS // 8, group_body, 0, unroll=False)

    for k in range(4):
        wait_out(_CPW - 1, k, k)


def kernel(x, emb_table):
    xf = x.reshape(-1).astype(jnp.int32)
    out = _emb_kernel(xf, emb_table, jnp.asarray(_PE))
    return out.reshape(_BATCH, _SEQ, _D)


# speedup vs baseline: 2.3380x; 1.5266x over previous
import functools

import numpy as np
import jax
import jax.numpy as jnp
from jax import lax
from jax.experimental import pallas as pl
from jax.experimental.pallas import tpu as pltpu
from jax.experimental.pallas import tpu_sc as plsc

_VOCAB = 100000
_D = 1024
_SEQ = 4096
_BATCH = 4
_SCALE = 32.0

_NC, _NS = 2, 16
_NW = _NC * _NS
_S_PER_W = _SEQ // _NW
_R = 8
_CPW = _S_PER_W // _R
_ROUNDS = _CPW * _BATCH
_LANES = 16
_NSLOT = 8
_PD = 4


def _pe_table() -> np.ndarray:
    pos = np.arange(_SEQ, dtype=np.float32)[:, None]
    div = np.exp(np.arange(0, _D, 2, dtype=np.float32) * (-np.log(10000.0) / _D))
    pe = np.zeros((_SEQ, _D), dtype=np.float32)
    pe[:, 0::2] = np.sin(pos * div)
    pe[:, 1::2] = np.cos(pos * div)
    return pe


_PE = _pe_table()

_mesh = plsc.VectorSubcoreMesh(core_axis_name="c", subcore_axis_name="s")


@functools.partial(
    pl.kernel,
    out_type=jax.ShapeDtypeStruct((_BATCH * _SEQ, _D), jnp.float32),
    mesh=_mesh,
    scratch_types=[
        pltpu.VMEM((_BATCH * _S_PER_W,), jnp.int32),
        pltpu.VMEM((_NSLOT, _R, _D), jnp.float32),
        pltpu.VMEM((2, _R, _D), jnp.float32),
        pltpu.SemaphoreType.DMA,
        pltpu.SemaphoreType.DMA,
        pltpu.SemaphoreType.DMA,
        pltpu.SemaphoreType.DMA,
        pltpu.SemaphoreType.DMA,
        pltpu.SemaphoreType.DMA,
        pltpu.SemaphoreType.DMA,
        pltpu.SemaphoreType.DMA,
        pltpu.SemaphoreType.DMA,
        pltpu.SemaphoreType.DMA,
        pltpu.SemaphoreType.DMA,
        pltpu.SemaphoreType.DMA,
        pltpu.SemaphoreType.DMA,
        pltpu.SemaphoreType.DMA,
        pltpu.SemaphoreType.DMA,
        pltpu.SemaphoreType.DMA,
        pltpu.SemaphoreType.DMA,
        pltpu.SemaphoreType.DMA,
    ],
)
def _emb_kernel(x_hbm, table_hbm, pe_hbm, out_hbm, idx_v, rows_v, pe_v,
                g0, g1, g2, g3, g4, g5, g6, g7,
                o0, o1, o2, o3, o4, o5, o6, o7, p0, p1):
    gsem = [g0, g1, g2, g3, g4, g5, g6, g7]
    osem = [o0, o1, o2, o3, o4, o5, o6, o7]
    psem = [p0, p1]

    wid = lax.axis_index("s") * _NC + lax.axis_index("c")
    s_base = wid * _S_PER_W

    for b in range(_BATCH):
        pltpu.sync_copy(x_hbm.at[pl.ds(b * _SEQ + s_base, _S_PER_W)],
                        idx_v.at[pl.ds(b * _S_PER_W, _S_PER_W)])

    def idx_slice(c, b):
        return idx_v.at[pl.ds(b * _S_PER_W + c * _R, _R)]

    def start_gather(c, b, slot):
        return pltpu.async_copy(table_hbm.at[idx_slice(c, b)],
                                rows_v.at[slot], gsem[slot])

    def wait_gather(c, b, slot):
        pltpu.make_async_copy(table_hbm.at[idx_slice(c, b)],
                              rows_v.at[slot], gsem[slot]).wait()

    def out_rows(c, b):
        return out_hbm.at[pl.ds(b * _SEQ + s_base + c * _R, _R)]

    def start_out(c, b, slot):
        return pltpu.async_copy(rows_v.at[slot], out_rows(c, b), osem[slot])

    def wait_out(c, b, slot):
        pltpu.make_async_copy(rows_v.at[slot], out_rows(c, b),
                              osem[slot]).wait()

    def pe_rows(c):
        return pe_hbm.at[pl.ds(s_base + c * _R, _R)]

    def start_pe(c, par):
        return pltpu.async_copy(pe_rows(c), pe_v.at[par], psem[par])

    def wait_pe(c, par):
        pltpu.make_async_copy(pe_rows(c), pe_v.at[par], psem[par]).wait()

    start_pe(0, 0)
    for r in range(_PD):
        start_gather(r // 4, r % 4, r)

    def group_body(gg, carry):
        for k in range(8):
            r = gg * 8 + k
            c = 2 * gg + k // 4
            b = k % 4
            slot = k
            par = (k // 4) % 2

            if k % 4 == 0:
                wait_pe(c, par)

                @pl.when(c < _CPW - 1)
                def _():
                    start_pe(c + 1, 1 - par)

            wait_gather(c, b, slot)

            def row_body(rr, car):
                for j in range(_D // _LANES):
                    sl = pl.ds(j * _LANES, _LANES)
                    rows_v[slot, rr, sl] = (rows_v[slot, rr, sl] * _SCALE
                                            + pe_v[par, rr, sl])
                return car

            lax.fori_loop(0, _R, row_body, 0, unroll=False)

            start_out(c, b, slot)

            k4 = k + _PD
            c4 = 2 * gg + k4 // 4
            b4 = k4 % 4
            slot4 = k4 % 8
            r4 = r + _PD

            @pl.when(r4 < _ROUNDS)
            def _():
                @pl.when(r >= _PD)
                def _():
                    wait_out(c4, b4, slot4)

                start_gather(c4, b4, slot4)

        return carry

    lax.fori_loop(0, _ROUNDS // 8, group_body, 0, unroll=False)

    for k in range(8):
        wait_out(_CPW - 2 + k // 4, k % 4, k)


def kernel(x, emb_table):
    xf = x.reshape(-1).astype(jnp.int32)
    out = _emb_kernel(xf, emb_table, jnp.asarray(_PE))
    return out.reshape(_BATCH, _SEQ, _D)


# speedup vs baseline: 2.4179x; 1.0342x over previous
import functools

import numpy as np
import jax
import jax.numpy as jnp
from jax import lax
from jax.experimental import pallas as pl
from jax.experimental.pallas import tpu as pltpu
from jax.experimental.pallas import tpu_sc as plsc

_VOCAB = 100000
_D = 1024
_SEQ = 4096
_BATCH = 4
_SCALE = 32.0

_NC, _NS = 2, 16
_NW = _NC * _NS
_S_PER_W = _SEQ // _NW
_R = 8
_CPW = _S_PER_W // _R
_ROUNDS = _CPW * _BATCH
_LANES = 16
_NSLOT = 8
_PD = 5
_GRP = 8


def _pe_table() -> np.ndarray:
    pos = np.arange(_SEQ, dtype=np.float32)[:, None]
    div = np.exp(np.arange(0, _D, 2, dtype=np.float32) * (-np.log(10000.0) / _D))
    pe = np.zeros((_SEQ, _D), dtype=np.float32)
    pe[:, 0::2] = np.sin(pos * div)
    pe[:, 1::2] = np.cos(pos * div)
    return pe


_PE = _pe_table()

_mesh = plsc.VectorSubcoreMesh(core_axis_name="c", subcore_axis_name="s")


@functools.partial(
    pl.kernel,
    out_type=jax.ShapeDtypeStruct((_BATCH * _SEQ, _D), jnp.float32),
    mesh=_mesh,
    scratch_types=(
        [pltpu.VMEM((_BATCH, _S_PER_W), jnp.int32),
         pltpu.VMEM((_NSLOT, _R, _D), jnp.float32),
         pltpu.VMEM((2, _R, _D), jnp.float32)]
        + [pltpu.SemaphoreType.DMA] * (_NSLOT + _NSLOT + 2)
    ),
)
def _emb_kernel(x_hbm, table_hbm, pe_hbm, out_hbm, idx_v, rows_v, pe_v,
                *sems):
    gsem = sems[:_NSLOT]
    osem = sems[_NSLOT:2 * _NSLOT]
    psem = sems[2 * _NSLOT:]

    wid = lax.axis_index("s") * _NC + lax.axis_index("c")
    s_base = wid * _S_PER_W

    pltpu.sync_copy(x_hbm.at[:, pl.ds(s_base, _S_PER_W)], idx_v)

    def idx_slice(c, b):
        return idx_v.at[b, pl.ds(c * _R, _R)]

    def start_gather(c, b, slot):
        return pltpu.async_copy(table_hbm.at[idx_slice(c, b)],
                                rows_v.at[slot], gsem[slot])

    def wait_gather(c, b, slot):
        pltpu.make_async_copy(table_hbm.at[idx_slice(c, b)],
                              rows_v.at[slot], gsem[slot]).wait()

    def out_rows(c, b):
        return out_hbm.at[pl.ds(b * _SEQ + s_base + c * _R, _R)]

    def start_out(c, b, slot):
        return pltpu.async_copy(rows_v.at[slot], out_rows(c, b), osem[slot])

    def wait_out(c, b, slot):
        pltpu.make_async_copy(rows_v.at[slot], out_rows(c, b),
                              osem[slot]).wait()

    def pe_rows(c):
        return pe_hbm.at[pl.ds(s_base + c * _R, _R)]

    def start_pe(c, par):
        return pltpu.async_copy(pe_rows(c), pe_v.at[par], psem[par])

    def wait_pe(c, par):
        pltpu.make_async_copy(pe_rows(c), pe_v.at[par], psem[par]).wait()

    start_pe(0, 0)
    for r in range(_PD):
        start_gather(r // 4, r % 4, r)

    def group_body(gg, carry):
        for k in range(_GRP):
            r = gg * _GRP + k
            c = (_GRP // 4) * gg + k // 4
            b = k % 4
            slot = k
            par = (k // 4) % 2

            if k % 4 == 0:
                wait_pe(c, par)

                @pl.when(c < _CPW - 1)
                def _():
                    start_pe(c + 1, 1 - par)

            wait_gather(c, b, slot)

            def row_body(rr, car):
                for j in range(_D // _LANES):
                    sl = pl.ds(j * _LANES, _LANES)
                    rows_v[slot, rr, sl] = (rows_v[slot, rr, sl] * _SCALE
                                            + pe_v[par, rr, sl])
                return car

            lax.fori_loop(0, _R, row_body, 0, unroll=False)

            start_out(c, b, slot)

            kp = k + _PD
            cp = (_GRP // 4) * gg + kp // 4
            bp = kp % 4
            slotp = kp % _NSLOT
            rp = r + _PD

            @pl.when(rp < _ROUNDS)
            def _():
                @pl.when(r >= _NSLOT - _PD)
                def _():
                    wait_out(cp, bp, slotp)

                start_gather(cp, bp, slotp)

        return carry

    lax.fori_loop(0, _ROUNDS // _GRP, group_body, 0, unroll=False)

    for rr in range(_ROUNDS - _NSLOT, _ROUNDS):
        wait_out(rr // 4, rr % 4, rr % _NSLOT)


def kernel(x, emb_table):
    xi = x.astype(jnp.int32)
    out = _emb_kernel(xi, emb_table, jnp.asarray(_PE))
    return out.reshape(_BATCH, _SEQ, _D)
